# baseline (speedup 1.0000x reference)
import functools

import jax
import jax.numpy as jnp
from jax.experimental import pallas as pl
from jax.experimental.pallas import tpu as pltpu


def _fused_kernel(H, W, Cin, Cout, Cr, x_ref, p_ref, o_ref, spad):
    HW = H * W
    PAD = 3 * W + 3

    r0 = 0
    pw = p_ref[r0:r0 + Cin, :]
    r0 += Cin
    pb = p_ref[r0:r0 + 1, :]
    r0 += 1
    b2 = p_ref[r0:r0 + 1, :]
    r0 += 1
    w2 = p_ref[r0:r0 + Cr, :]
    r0 += Cr
    wk = p_ref[r0:r0 + 49, :]
    r0 += 49
    w1 = p_ref[r0:r0 + Cout, 0:Cr]
    r0 += Cout
    b1 = p_ref[r0:r0 + 1, 0:Cr]
    r0 += 1
    bk = p_ref[r0:r0 + 1, 0:1]

    xb = x_ref[0]
    ones_row = jnp.ones((1, HW), jnp.float32)
    xb_aug = jnp.concatenate([xb, ones_row], axis=0)
    pw_aug = jnp.concatenate([pw, pb], axis=0)

    dn00 = (((0,), (0,)), ((), ()))
    yT = jax.lax.dot_general(
        pw_aug.astype(jnp.bfloat16), xb_aug.astype(jnp.bfloat16),
        dn00, preferred_element_type=jnp.float32)

    meanC = jnp.sum(yT, axis=1, keepdims=True) * (1.0 / float(HW))
    one11 = jnp.ones((1, 1), jnp.float32)

    w1_aug = jnp.concatenate([w1, b1], axis=0)
    mean_aug = jnp.concatenate([meanC, one11], axis=0)
    z1 = jnp.maximum(jax.lax.dot_general(
        w1_aug, mean_aug, dn00, preferred_element_type=jnp.float32), 0.0)
    w2_aug = jnp.concatenate([w2, b2], axis=0)
    z1_aug = jnp.concatenate([z1, one11], axis=0)
    attC = jax.nn.sigmoid(jax.lax.dot_general(
        w2_aug, z1_aug, dn00, preferred_element_type=jnp.float32))

    xcw = yT * attC

    R = jnp.dot(wk.astype(jnp.bfloat16), xcw.astype(jnp.bfloat16),
                preferred_element_type=jnp.float32)

    spad[:, 0:PAD] = jnp.zeros((49, PAD), jnp.float32)
    spad[:, PAD + HW:] = jnp.zeros((49, PAD), jnp.float32)
    spad[:, PAD:PAD + HW] = R

    wl = jax.lax.broadcasted_iota(jnp.int32, (1, HW), 1)
    wl = (wl & (W - 1)) if (W & (W - 1)) == 0 else (wl % W)

    acc = jnp.zeros((1, HW), jnp.float32) + bk
    for kj in range(7):
        inner = None
        for ki in range(7):
            t = ki * 7 + kj
            off = PAD + (ki - 3) * W + (kj - 3)
            sl = spad[t:t + 1, off:off + HW]
            inner = sl if inner is None else inner + sl
        if kj < 3:
            inner = jnp.where(wl >= (3 - kj), inner, 0.0)
        elif kj > 3:
            inner = jnp.where(wl < (W + 3 - kj), inner, 0.0)
        acc = acc + inner

    att_s = jax.nn.sigmoid(acc)
    o_ref[0] = xcw * att_s


def kernel(x, proj_w, proj_b, se_w1, se_b1, se_w2, se_b2, sp_w, sp_b):
    B, Cin, H, W = x.shape
    Cout = proj_w.shape[1]
    Cr = se_w1.shape[1]
    HW = H * W
    PAD = 3 * W + 3

    x3 = x.reshape(B, Cin, HW)

    packed = jnp.concatenate([
        proj_w,
        proj_b.reshape(1, Cout),
        se_b2.reshape(1, Cout),
        se_w2,
        sp_w.reshape(49, Cout),
        jnp.pad(se_w1, ((0, 0), (0, Cout - Cr))),
        jnp.pad(se_b1.reshape(1, Cr), ((0, 0), (0, Cout - Cr))),
        jnp.pad(sp_b.reshape(1, 1), ((0, 0), (0, Cout - 1))),
    ], axis=0)
    n_rows = Cin + 2 + Cr + 49 + Cout + 2

    out = pl.pallas_call(
        functools.partial(_fused_kernel, H, W, Cin, Cout, Cr),
        out_shape=jax.ShapeDtypeStruct((B, Cout, HW), jnp.float32),
        grid=(B,),
        in_specs=[
            pl.BlockSpec((1, Cin, HW), lambda b: (b, 0, 0)),
            pl.BlockSpec((n_rows, Cout), lambda b: (0, 0)),
        ],
        out_specs=pl.BlockSpec((1, Cout, HW), lambda b: (b, 0, 0)),
        scratch_shapes=[pltpu.VMEM((49, 2 * PAD + HW), jnp.float32)],
        compiler_params=pltpu.CompilerParams(
            dimension_semantics=("parallel",),
            vmem_limit_bytes=64 * 1024 * 1024),
        cost_estimate=pl.CostEstimate(
            flops=2 * B * HW * Cout * (Cin + 49),
            transcendentals=B * (HW + Cout),
            bytes_accessed=4 * (B * HW * (Cin + Cout) + n_rows * Cout),
        ),
    )(x3, packed)

    return out.reshape(B, Cout, H, W)


# speedup vs baseline: 1.0550x; 1.0550x over previous
import functools

import jax
import jax.numpy as jnp
from jax.experimental import pallas as pl
from jax.experimental.pallas import tpu as pltpu


def _fused_kernel(H, W, Cin, Cout, Cr, x_ref, pw_ref, pb_ref, w1_ref, b1_ref,
                  w2_ref, b2_ref, wk_ref, bk_ref, o_ref, spad):
    HW = H * W
    PAD = 3 * W + 3

    pw = pw_ref[...]
    pb = pb_ref[...]
    w1 = w1_ref[...]
    b1 = b1_ref[...]
    w2 = w2_ref[...]
    b2 = b2_ref[...]
    wk = wk_ref[...].reshape(49, Cout)
    bk = bk_ref[...]

    xb = x_ref[0]
    ones_row = jnp.ones((1, HW), jnp.float32)
    xb_aug = jnp.concatenate([xb, ones_row], axis=0)
    pw_aug = jnp.concatenate([pw, pb], axis=0)

    dn00 = (((0,), (0,)), ((), ()))
    yT = jax.lax.dot_general(
        pw_aug.astype(jnp.bfloat16), xb_aug.astype(jnp.bfloat16),
        dn00, preferred_element_type=jnp.float32)

    meanC = jnp.sum(yT, axis=1, keepdims=True) * (1.0 / float(HW))
    one11 = jnp.ones((1, 1), jnp.float32)

    w1_aug = jnp.concatenate([w1, b1], axis=0)
    mean_aug = jnp.concatenate([meanC, one11], axis=0)
    z1 = jnp.maximum(jax.lax.dot_general(
        w1_aug, mean_aug, dn00, preferred_element_type=jnp.float32), 0.0)
    w2_aug = jnp.concatenate([w2, b2], axis=0)
    z1_aug = jnp.concatenate([z1, one11], axis=0)
    attC = jax.nn.sigmoid(jax.lax.dot_general(
        w2_aug, z1_aug, dn00, preferred_element_type=jnp.float32))

    xcw = yT * attC

    R = jnp.dot(wk.astype(jnp.bfloat16), xcw.astype(jnp.bfloat16),
                preferred_element_type=jnp.float32)

    spad[:, 0:PAD] = jnp.zeros((49, PAD), jnp.float32)
    spad[:, PAD + HW:] = jnp.zeros((49, PAD), jnp.float32)
    spad[:, PAD:PAD + HW] = R

    wl = jax.lax.broadcasted_iota(jnp.int32, (1, HW), 1)
    wl = (wl & (W - 1)) if (W & (W - 1)) == 0 else (wl % W)

    acc = jnp.zeros((1, HW), jnp.float32) + bk
    for kj in range(7):
        inner = None
        for ki in range(7):
            t = ki * 7 + kj
            off = PAD + (ki - 3) * W + (kj - 3)
            sl = spad[t:t + 1, off:off + HW]
            inner = sl if inner is None else inner + sl
        if kj < 3:
            inner = jnp.where(wl >= (3 - kj), inner, 0.0)
        elif kj > 3:
            inner = jnp.where(wl < (W + 3 - kj), inner, 0.0)
        acc = acc + inner

    att_s = jax.nn.sigmoid(acc)
    o_ref[0] = xcw * att_s


def kernel(x, proj_w, proj_b, se_w1, se_b1, se_w2, se_b2, sp_w, sp_b):
    B, Cin, H, W = x.shape
    Cout = proj_w.shape[1]
    Cr = se_w1.shape[1]
    HW = H * W
    PAD = 3 * W + 3

    x3 = x.reshape(B, Cin, HW)
    fixed = lambda b: (0, 0)

    out = pl.pallas_call(
        functools.partial(_fused_kernel, H, W, Cin, Cout, Cr),
        out_shape=jax.ShapeDtypeStruct((B, Cout, HW), jnp.float32),
        grid=(B,),
        in_specs=[
            pl.BlockSpec((1, Cin, HW), lambda b: (b, 0, 0)),
            pl.BlockSpec((Cin, Cout), fixed),
            pl.BlockSpec((1, Cout), fixed),
            pl.BlockSpec((Cout, Cr), fixed),
            pl.BlockSpec((1, Cr), fixed),
            pl.BlockSpec((Cr, Cout), fixed),
            pl.BlockSpec((1, Cout), fixed),
            pl.BlockSpec((7, 7, Cout), lambda b: (0, 0, 0)),
            pl.BlockSpec((1, 1), fixed),
        ],
        out_specs=pl.BlockSpec((1, Cout, HW), lambda b: (b, 0, 0)),
        scratch_shapes=[pltpu.VMEM((49, 2 * PAD + HW), jnp.float32)],
        compiler_params=pltpu.CompilerParams(
            dimension_semantics=("parallel",),
            vmem_limit_bytes=64 * 1024 * 1024),
        cost_estimate=pl.CostEstimate(
            flops=2 * B * HW * Cout * (Cin + 49),
            transcendentals=B * (HW + Cout),
            bytes_accessed=4 * (B * HW * (Cin + Cout) + 3 * Cout * Cin),
        ),
    )(x3, proj_w, proj_b.reshape(1, Cout), se_w1, se_b1.reshape(1, Cr),
      se_w2, se_b2.reshape(1, Cout), sp_w, sp_b.reshape(1, 1))

    return out.reshape(B, Cout, H, W)


# speedup vs baseline: 1.1506x; 1.0906x over previous
import functools

import jax
import jax.numpy as jnp
from jax.experimental import pallas as pl
from jax.experimental.pallas import tpu as pltpu


def _fused_kernel(H, W, Cin, Cout, Cr, x_ref, pw_ref, pb_ref, w1_ref, b1_ref,
                  w2_ref, b2_ref, wk_ref, bk_ref, o_ref, spad):
    HW = H * W
    PAD = 3 * W + 3

    pw = pw_ref[...]
    pb = pb_ref[...]
    w1 = w1_ref[...]
    b1 = b1_ref[...]
    w2 = w2_ref[...]
    b2 = b2_ref[...]
    wk = wk_ref[...].reshape(49, Cout)
    bk = bk_ref[...]

    xb = x_ref[0]
    ones_row = jnp.ones((1, HW), jnp.float32)
    xb_aug = jnp.concatenate([xb, ones_row], axis=0)
    pw_aug = jnp.concatenate([pw, pb], axis=0)

    dn00 = (((0,), (0,)), ((), ()))
    yT = jax.lax.dot_general(
        pw_aug.astype(jnp.bfloat16), xb_aug.astype(jnp.bfloat16),
        dn00, preferred_element_type=jnp.float32)

    meanC = jnp.sum(yT, axis=1, keepdims=True) * (1.0 / float(HW))
    one11 = jnp.ones((1, 1), jnp.float32)

    w1_aug = jnp.concatenate([w1, b1], axis=0)
    mean_aug = jnp.concatenate([meanC, one11], axis=0)
    z1 = jnp.maximum(jax.lax.dot_general(
        w1_aug, mean_aug, dn00, preferred_element_type=jnp.float32), 0.0)
    w2_aug = jnp.concatenate([w2, b2], axis=0)
    z1_aug = jnp.concatenate([z1, one11], axis=0)
    attC = jax.nn.sigmoid(jax.lax.dot_general(
        w2_aug, z1_aug, dn00, preferred_element_type=jnp.float32))

    xcw = yT * attC

    R = jnp.dot(wk.astype(jnp.bfloat16), xcw.astype(jnp.bfloat16),
                preferred_element_type=jnp.float32)

    spad[:, 0:PAD] = jnp.zeros((49, PAD), jnp.float32)
    spad[:, PAD + HW:] = jnp.zeros((49, PAD), jnp.float32)
    spad[:, PAD:PAD + HW] = R

    wl = jax.lax.broadcasted_iota(jnp.int32, (1, HW), 1)
    wl = (wl & (W - 1)) if (W & (W - 1)) == 0 else (wl % W)

    acc = jnp.zeros((1, HW), jnp.float32) + bk
    for kj in range(7):
        inner = None
        for ki in range(7):
            t = ki * 7 + kj
            off = PAD + (ki - 3) * W + (kj - 3)
            sl = spad[t:t + 1, off:off + HW]
            inner = sl if inner is None else inner + sl
        if kj < 3:
            inner = jnp.where(wl >= (3 - kj), inner, 0.0)
        elif kj > 3:
            inner = jnp.where(wl < (W + 3 - kj), inner, 0.0)
        acc = acc + inner

    att_s = jax.nn.sigmoid(acc)
    o_ref[0] = (xcw * att_s).astype(jnp.bfloat16)


def kernel(x, proj_w, proj_b, se_w1, se_b1, se_w2, se_b2, sp_w, sp_b):
    B, Cin, H, W = x.shape
    Cout = proj_w.shape[1]
    Cr = se_w1.shape[1]
    HW = H * W
    PAD = 3 * W + 3

    x3 = x.reshape(B, Cin, HW)
    fixed = lambda b: (0, 0)

    out = pl.pallas_call(
        functools.partial(_fused_kernel, H, W, Cin, Cout, Cr),
        out_shape=jax.ShapeDtypeStruct((B, Cout, HW), jnp.bfloat16),
        grid=(B,),
        in_specs=[
            pl.BlockSpec((1, Cin, HW), lambda b: (b, 0, 0)),
            pl.BlockSpec((Cin, Cout), fixed),
            pl.BlockSpec((1, Cout), fixed),
            pl.BlockSpec((Cout, Cr), fixed),
            pl.BlockSpec((1, Cr), fixed),
            pl.BlockSpec((Cr, Cout), fixed),
            pl.BlockSpec((1, Cout), fixed),
            pl.BlockSpec((7, 7, Cout), lambda b: (0, 0, 0)),
            pl.BlockSpec((1, 1), fixed),
        ],
        out_specs=pl.BlockSpec((1, Cout, HW), lambda b: (b, 0, 0)),
        scratch_shapes=[pltpu.VMEM((49, 2 * PAD + HW), jnp.float32)],
        compiler_params=pltpu.CompilerParams(
            dimension_semantics=("parallel",),
            vmem_limit_bytes=64 * 1024 * 1024),
        cost_estimate=pl.CostEstimate(
            flops=2 * B * HW * Cout * (Cin + 49),
            transcendentals=B * (HW + Cout),
            bytes_accessed=4 * (B * HW * (Cin + Cout) + 3 * Cout * Cin),
        ),
    )(x3, proj_w, proj_b.reshape(1, Cout), se_w1, se_b1.reshape(1, Cr),
      se_w2, se_b2.reshape(1, Cout), sp_w, sp_b.reshape(1, 1))

    return out.reshape(B, Cout, H, W).astype(jnp.float32)


# speedup vs baseline: 1.1573x; 1.0058x over previous
import functools

import jax
import jax.numpy as jnp
from jax.experimental import pallas as pl
from jax.experimental.pallas import tpu as pltpu


def _fused_kernel(H, W, Cin, Cout, Cr, x_ref, pw_ref, pb_ref, w1_ref, b1_ref,
                  w2_ref, b2_ref, wk_ref, bk_ref, o_ref, spad):
    HW = H * W
    PAD = 3 * W + 3

    pw = pw_ref[...]
    pb = pb_ref[...]
    w1 = w1_ref[...]
    b1 = b1_ref[...]
    w2 = w2_ref[...]
    b2 = b2_ref[...]
    wk = wk_ref[...].reshape(49, Cout)
    bk = bk_ref[...]

    xb = x_ref[0]
    ones_row = jnp.ones((1, HW), jnp.float32)
    xb_aug = jnp.concatenate([xb, ones_row], axis=0)
    pw_aug = jnp.concatenate([pw, pb], axis=0)

    dn00 = (((0,), (0,)), ((), ()))
    yT = jax.lax.dot_general(
        pw_aug.astype(jnp.bfloat16), xb_aug.astype(jnp.bfloat16),
        dn00, preferred_element_type=jnp.float32)

    meanC = jnp.sum(yT, axis=1, keepdims=True) * (1.0 / float(HW))
    one11 = jnp.ones((1, 1), jnp.float32)

    w1_aug = jnp.concatenate([w1, b1], axis=0)
    mean_aug = jnp.concatenate([meanC, one11], axis=0)
    z1 = jnp.maximum(jax.lax.dot_general(
        w1_aug, mean_aug, dn00, preferred_element_type=jnp.float32), 0.0)
    w2_aug = jnp.concatenate([w2, b2], axis=0)
    z1_aug = jnp.concatenate([z1, one11], axis=0)
    attC = jax.nn.sigmoid(jax.lax.dot_general(
        w2_aug, z1_aug, dn00, preferred_element_type=jnp.float32))
    att_row = jax.nn.sigmoid(jax.lax.dot_general(
        z1_aug, w2_aug, dn00, preferred_element_type=jnp.float32))

    R = jnp.dot((wk * att_row).astype(jnp.bfloat16), yT.astype(jnp.bfloat16),
                preferred_element_type=jnp.float32)

    spad[:, 0:PAD] = jnp.zeros((49, PAD), jnp.float32)
    spad[:, PAD + HW:] = jnp.zeros((49, PAD), jnp.float32)
    spad[:, PAD:PAD + HW] = R

    wl = jax.lax.broadcasted_iota(jnp.int32, (1, HW), 1)
    wl = (wl & (W - 1)) if (W & (W - 1)) == 0 else (wl % W)

    acc = jnp.zeros((1, HW), jnp.float32) + bk
    for kj in range(7):
        inner = None
        for ki in range(7):
            t = ki * 7 + kj
            off = PAD + (ki - 3) * W + (kj - 3)
            sl = spad[t:t + 1, off:off + HW]
            inner = sl if inner is None else inner + sl
        if kj < 3:
            inner = jnp.where(wl >= (3 - kj), inner, 0.0)
        elif kj > 3:
            inner = jnp.where(wl < (W + 3 - kj), inner, 0.0)
        acc = acc + inner

    att_s = jax.nn.sigmoid(acc)
    o_ref[0] = ((yT * attC) * att_s).astype(jnp.bfloat16)


def kernel(x, proj_w, proj_b, se_w1, se_b1, se_w2, se_b2, sp_w, sp_b):
    B, Cin, H, W = x.shape
    Cout = proj_w.shape[1]
    Cr = se_w1.shape[1]
    HW = H * W
    PAD = 3 * W + 3

    x3 = x.reshape(B, Cin, HW)
    fixed = lambda b: (0, 0)

    out = pl.pallas_call(
        functools.partial(_fused_kernel, H, W, Cin, Cout, Cr),
        out_shape=jax.ShapeDtypeStruct((B, Cout, HW), jnp.bfloat16),
        grid=(B,),
        in_specs=[
            pl.BlockSpec((1, Cin, HW), lambda b: (b, 0, 0)),
            pl.BlockSpec((Cin, Cout), fixed),
            pl.BlockSpec((1, Cout), fixed),
            pl.BlockSpec((Cout, Cr), fixed),
            pl.BlockSpec((1, Cr), fixed),
            pl.BlockSpec((Cr, Cout), fixed),
            pl.BlockSpec((1, Cout), fixed),
            pl.BlockSpec((7, 7, Cout), lambda b: (0, 0, 0)),
            pl.BlockSpec((1, 1), fixed),
        ],
        out_specs=pl.BlockSpec((1, Cout, HW), lambda b: (b, 0, 0)),
        scratch_shapes=[pltpu.VMEM((49, 2 * PAD + HW), jnp.float32)],
        compiler_params=pltpu.CompilerParams(
            dimension_semantics=("parallel",),
            vmem_limit_bytes=64 * 1024 * 1024),
        cost_estimate=pl.CostEstimate(
            flops=2 * B * HW * Cout * (Cin + 49),
            transcendentals=B * (HW + Cout),
            bytes_accessed=4 * (B * HW * (Cin + Cout) + 3 * Cout * Cin),
        ),
    )(x3, proj_w, proj_b.reshape(1, Cout), se_w1, se_b1.reshape(1, Cr),
      se_w2, se_b2.reshape(1, Cout), sp_w, sp_b.reshape(1, 1))

    return out.reshape(B, Cout, H, W).astype(jnp.float32)


# speedup vs baseline: 1.2106x; 1.0461x over previous
import functools

import jax
import jax.numpy as jnp
from jax.experimental import pallas as pl
from jax.experimental.pallas import tpu as pltpu


def _one_batch(H, W, Cout, xb, pw_aug, w1_aug, w2_aug, wk, bk, o_slot, spad):
    HW = H * W
    PAD = 3 * W + 3
    dn00 = (((0,), (0,)), ((), ()))

    ones_row = jnp.ones((1, HW), jnp.float32)
    xb_aug = jnp.concatenate([xb, ones_row], axis=0)

    yT = jax.lax.dot_general(
        pw_aug, xb_aug.astype(jnp.bfloat16),
        dn00, preferred_element_type=jnp.float32)

    meanC = jnp.sum(yT, axis=1, keepdims=True) * (1.0 / float(HW))
    one11 = jnp.ones((1, 1), jnp.float32)

    mean_aug = jnp.concatenate([meanC, one11], axis=0)
    z1 = jnp.maximum(jax.lax.dot_general(
        w1_aug, mean_aug, dn00, preferred_element_type=jnp.float32), 0.0)
    z1_aug = jnp.concatenate([z1, one11], axis=0)
    attC = jax.nn.sigmoid(jax.lax.dot_general(
        w2_aug, z1_aug, dn00, preferred_element_type=jnp.float32))
    att_row = jax.nn.sigmoid(jax.lax.dot_general(
        z1_aug, w2_aug, dn00, preferred_element_type=jnp.float32))

    R = jnp.dot((wk * att_row).astype(jnp.bfloat16), yT.astype(jnp.bfloat16),
                preferred_element_type=jnp.float32)

    spad[:, 0:PAD] = jnp.zeros((49, PAD), jnp.float32)
    spad[:, PAD + HW:] = jnp.zeros((49, PAD), jnp.float32)
    spad[:, PAD:PAD + HW] = R

    wl = jax.lax.broadcasted_iota(jnp.int32, (1, HW), 1)
    wl = (wl & (W - 1)) if (W & (W - 1)) == 0 else (wl % W)

    acc = jnp.zeros((1, HW), jnp.float32) + bk
    for kj in range(7):
        inner = None
        for ki in range(7):
            t = ki * 7 + kj
            off = PAD + (ki - 3) * W + (kj - 3)
            sl = spad[t:t + 1, off:off + HW]
            inner = sl if inner is None else inner + sl
        if kj < 3:
            inner = jnp.where(wl >= (3 - kj), inner, 0.0)
        elif kj > 3:
            inner = jnp.where(wl < (W + 3 - kj), inner, 0.0)
        acc = acc + inner

    att_s = jax.nn.sigmoid(acc)
    o_slot[...] = ((yT * attC) * att_s).astype(jnp.bfloat16)


def _fused_kernel(H, W, Cin, Cout, Cr, NB, x_ref, pw_ref, pb_ref, w1_ref,
                  b1_ref, w2_ref, b2_ref, wk_ref, bk_ref, o_ref, spad):
    pw_aug = jnp.concatenate([pw_ref[...], pb_ref[...]],
                             axis=0).astype(jnp.bfloat16)
    w1_aug = jnp.concatenate([w1_ref[...], b1_ref[...]], axis=0)
    w2_aug = jnp.concatenate([w2_ref[...], b2_ref[...]], axis=0)
    wk = wk_ref[...].reshape(49, Cout)
    bk = bk_ref[...]

    for i in range(NB):
        _one_batch(H, W, Cout, x_ref[i], pw_aug, w1_aug, w2_aug, wk, bk,
                   o_ref.at[i], spad.at[i])


def kernel(x, proj_w, proj_b, se_w1, se_b1, se_w2, se_b2, sp_w, sp_b):
    B, Cin, H, W = x.shape
    Cout = proj_w.shape[1]
    Cr = se_w1.shape[1]
    HW = H * W
    PAD = 3 * W + 3
    NB = 2 if B % 2 == 0 else 1

    x3 = x.reshape(B, Cin, HW)
    fixed = lambda b: (0, 0)

    out = pl.pallas_call(
        functools.partial(_fused_kernel, H, W, Cin, Cout, Cr, NB),
        out_shape=jax.ShapeDtypeStruct((B, Cout, HW), jnp.bfloat16),
        grid=(B // NB,),
        in_specs=[
            pl.BlockSpec((NB, Cin, HW), lambda b: (b, 0, 0)),
            pl.BlockSpec((Cin, Cout), fixed),
            pl.BlockSpec((1, Cout), fixed),
            pl.BlockSpec((Cout, Cr), fixed),
            pl.BlockSpec((1, Cr), fixed),
            pl.BlockSpec((Cr, Cout), fixed),
            pl.BlockSpec((1, Cout), fixed),
            pl.BlockSpec((7, 7, Cout), lambda b: (0, 0, 0)),
            pl.BlockSpec((1, 1), fixed),
        ],
        out_specs=pl.BlockSpec((NB, Cout, HW), lambda b: (b, 0, 0)),
        scratch_shapes=[pltpu.VMEM((NB, 49, 2 * PAD + HW), jnp.float32)],
        compiler_params=pltpu.CompilerParams(
            dimension_semantics=("parallel",),
            vmem_limit_bytes=64 * 1024 * 1024),
        cost_estimate=pl.CostEstimate(
            flops=2 * B * HW * Cout * (Cin + 49),
            transcendentals=B * (HW + Cout),
            bytes_accessed=4 * (B * HW * Cin + 3 * Cout * Cin)
            + 2 * B * HW * Cout,
        ),
    )(x3, proj_w, proj_b.reshape(1, Cout), se_w1, se_b1.reshape(1, Cr),
      se_w2, se_b2.reshape(1, Cout), sp_w, sp_b.reshape(1, 1))

    return out.reshape(B, Cout, H, W).astype(jnp.float32)
